# baseline (speedup 1.0000x reference)
import jax
import jax.numpy as jnp
from jax.experimental import pallas as pl
from jax.experimental.pallas import tpu as pltpu

_N_ROW_BLOCKS = 8
_VMEM_LIMIT_BYTES = 48 * 1024 * 1024


def _body(x_ref, w_ref, b_ref, o_hbm, acc, sem_o):
    k = pl.program_id(0)
    i = pl.program_id(1)
    n_m = acc.shape[0]
    tm = acc.shape[1]

    def cp_o(chunk):
        return pltpu.make_async_copy(
            acc.at[chunk], o_hbm.at[pl.ds(chunk * tm, tm), :], sem_o.at[chunk]
        )

    part = jnp.dot(x_ref[...], w_ref[...], preferred_element_type=jnp.float32)

    @pl.when(k == 0)
    def _():
        acc[i] = part

    @pl.when(k == 1)
    def _():
        acc[i] = acc[i] + part + b_ref[...]
        cp_o(i).start()

    @pl.when((k == 1) & (i == n_m - 1))
    def _():
        for j in range(n_m):
            cp_o(j).wait()


def kernel(x, w_packed, b_packed):
    B, F = x.shape
    C = w_packed.shape[1]
    n_m = _N_ROW_BLOCKS if B % _N_ROW_BLOCKS == 0 else 1
    tm = B // n_m
    k2 = F // 2

    cost = pl.CostEstimate(
        flops=2 * B * C * F,
        transcendentals=0,
        bytes_accessed=4 * (B * F + F * C + B * C),
    )
    return pl.pallas_call(
        _body,
        out_shape=jax.ShapeDtypeStruct((B, C), jnp.float32),
        grid=(2, n_m),
        in_specs=[
            pl.BlockSpec((tm, k2), lambda k, i: (i, k)),
            pl.BlockSpec((k2, C), lambda k, i: (k, 0)),
            pl.BlockSpec((1, C), lambda k, i: (0, 0)),
        ],
        out_specs=pl.BlockSpec(memory_space=pl.ANY),
        scratch_shapes=[
            pltpu.VMEM((n_m, tm, C), jnp.float32),
            pltpu.SemaphoreType.DMA((n_m,)),
        ],
        compiler_params=pltpu.CompilerParams(
            dimension_semantics=("arbitrary", "arbitrary"),
            vmem_limit_bytes=_VMEM_LIMIT_BYTES,
        ),
        cost_estimate=cost,
    )(x, w_packed, b_packed)


# speedup vs baseline: 1.2085x; 1.2085x over previous
import jax
import jax.numpy as jnp
from jax.experimental import pallas as pl
from jax.experimental.pallas import tpu as pltpu

_TM = 1024
_VMEM_LIMIT_BYTES = 48 * 1024 * 1024


def _body(x_ref, w_ref, b_ref, o_ref):
    o_ref[...] = (
        jnp.dot(x_ref[...], w_ref[...], preferred_element_type=jnp.float32)
        + b_ref[...]
    )


def kernel(x, w_packed, b_packed):
    B, F = x.shape
    C = w_packed.shape[1]

    tm = _TM if B % _TM == 0 else B
    grid = (B // tm,)

    cost = pl.CostEstimate(
        flops=2 * B * C * F,
        transcendentals=0,
        bytes_accessed=4 * (B * F + F * C + B * C),
    )
    return pl.pallas_call(
        _body,
        out_shape=jax.ShapeDtypeStruct((B, C), jnp.float32),
        grid=grid,
        in_specs=[
            pl.BlockSpec((tm, F), lambda i: (i, 0)),
            pl.BlockSpec((F, C), lambda i: (0, 0)),
            pl.BlockSpec((1, C), lambda i: (0, 0)),
        ],
        out_specs=pl.BlockSpec((tm, C), lambda i: (i, 0)),
        compiler_params=pltpu.CompilerParams(
            dimension_semantics=("parallel",),
            vmem_limit_bytes=_VMEM_LIMIT_BYTES,
        ),
        cost_estimate=cost,
    )(x, w_packed, b_packed)
